# baseline (speedup 1.0000x reference)
import jax
import jax.numpy as jnp
from jax import lax
from jax.experimental import pallas as pl
from jax.experimental.pallas import tpu as pltpu
from jax.experimental.pallas import tpu_sc as plsc
from jax._src.pallas import mpmd

_M = 1_000_000
_D = 64
_B = 16384

_NC = 2
_NS = 16
_NW = _NC * _NS

_TM = 128
_NT = _M // _TM
_TAIL = _M - _NT * _TM
_TPW = (_NT + _NW - 1) // _NW
_NBUF = 3

_OFFQ = (_NT + 2 + 1023) // 1024

_CH = 128
_NCH = _B // _NW // _CH

_mesh = plsc.VectorSubcoreMesh(
    core_axis_name="c", subcore_axis_name="s", num_cores=_NC, num_subcores=_NS)


def _splat(ref, idx_scalars):
    return plsc.load_gather(
        ref, [jnp.full((16,), i, jnp.int32) for i in idx_scalars])


def _scalar(vec):
    return jnp.squeeze(lax.slice(vec, (0,), (1,)))


def _mem_body(memT_hbm, valsT_hbm, sidx_hbm, out_memT,
              tile_v, vals_v, sidx_v, sem_in, sem_out):
    wid = lax.axis_index("s") * _NC + lax.axis_index("c")
    t0 = wid * _TPW
    nt = jnp.minimum(_TPW, _NT - t0)

    pltpu.sync_copy(sidx_hbm, sidx_v)

    def read_sidx(k):
        kc = jnp.minimum(k, _B - 1)
        return _scalar(_splat(sidx_v, (kc >> 10, (kc >> 7) & 7, kc & 127)))

    target = t0 * _TM
    lo = jnp.int32(0)
    hi = jnp.int32(_B)
    for _ in range(14):
        mid = (lo + hi) >> 1
        below = read_sidx(mid) < target
        lo = jnp.where(below, mid + 1, lo)
        hi = jnp.where(below, hi, mid)

    def col_base(t):
        return pl.multiple_of(t * _TM, _TM)

    def start_in(j, buf):
        pltpu.async_copy(memT_hbm.at[:, pl.ds(col_base(t0 + j), _TM)],
                         tile_v.at[buf], sem_in.at[buf])

    def patch(buf, t, carry):
        limit = (t + 1) * _TM
        def cond(c):
            k, _ = c
            return (k < _B) & (read_sidx(k) < limit)
        def step(c):
            k, gv = c
            g_new = k >> 7
            @pl.when(g_new != gv)
            def _():
                pltpu.sync_copy(valsT_hbm.at[g_new], vals_v)
            lv = k - g_new * _TM
            rel = read_sidx(k) - t * _TM
            for p in range(_D // 16):
                dvec = lax.iota(jnp.int32, 16) + 16 * p
                col = plsc.load_gather(
                    vals_v, [dvec, jnp.full((16,), lv, jnp.int32)])
                plsc.store_scatter(
                    tile_v.at[buf], [dvec, jnp.full((16,), rel, jnp.int32)],
                    col)
            return k + 1, g_new
        return lax.while_loop(cond, step, carry)

    def body(j, carry):
        buf = j % _NBUF
        @pl.when(j == 0)
        def _():
            start_in(0, 0)
        nxt = (j + 1) % _NBUF
        @pl.when((j + 1 < nt) & (j >= _NBUF - 1))
        def _():
            pltpu.make_async_copy(
                tile_v.at[nxt],
                out_memT.at[:, pl.ds(col_base(t0 + j + 1 - _NBUF), _TM)],
                sem_out.at[nxt]).wait()
        @pl.when(j + 1 < nt)
        def _():
            start_in(j + 1, nxt)

        pltpu.make_async_copy(
            memT_hbm.at[:, pl.ds(col_base(t0 + j), _TM)],
            tile_v.at[buf], sem_in.at[buf]).wait()

        carry = patch(buf, t0 + j, carry)

        pltpu.async_copy(tile_v.at[buf],
                         out_memT.at[:, pl.ds(col_base(t0 + j), _TM)],
                         sem_out.at[buf])
        return carry

    carry = lax.fori_loop(0, nt, body, (lo, jnp.int32(-1)))

    for i in range(_NBUF):
        @pl.when(nt - 1 - i >= 0)
        def _():
            jj = nt - 1 - i
            pltpu.make_async_copy(
                tile_v.at[jj % _NBUF],
                out_memT.at[:, pl.ds(col_base(t0 + jj), _TM)],
                sem_out.at[jj % _NBUF]).wait()


_scatter_mem = mpmd._mpmd_map(
    [(_mesh, _mem_body)],
    [jax.ShapeDtypeStruct((_D, _M), jnp.float32)],
    scratch_types=[
        pltpu.VMEM((_NBUF, _D, _TM), jnp.float32),
        pltpu.VMEM((_D, _TM), jnp.float32),
        pltpu.VMEM((_B // 1024, 8, 128), jnp.int32),
        pltpu.SemaphoreType.DMA((_NBUF,)),
        pltpu.SemaphoreType.DMA((_NBUF,)),
    ],
    compiler_params=pltpu.CompilerParams(needs_layout_passes=False),
    name="tracker_scatter_mem",
)


def _frames_body(frames_hbm, idx_hbm, fvals_hbm, out_frames, idx_v, fv_v, sem):
    del frames_hbm
    wid = lax.axis_index("s") * _NC + lax.axis_index("c")
    pltpu.sync_copy(idx_hbm.at[wid], idx_v)
    pltpu.sync_copy(fvals_hbm.at[wid], fv_v)
    copies = []
    for j in range(_NCH):
        copies.append(
            pltpu.async_copy(fv_v.at[j], out_frames.at[idx_v.at[j]], sem))
    for cp in copies:
        cp.wait()


_scatter_frames = mpmd._mpmd_map(
    [(_mesh, _frames_body)],
    [jax.ShapeDtypeStruct((_M,), jnp.int32)],
    input_output_aliases={0: 0},
    scratch_types=[
        pltpu.VMEM((_NCH, _CH), jnp.int32),
        pltpu.VMEM((_NCH, _CH), jnp.int32),
        pltpu.SemaphoreType.DMA,
    ],
    compiler_params=pltpu.CompilerParams(use_tc_tiling_on_sc=False),
    name="tracker_scatter_frames",
)


def kernel(mem, vals, matches, frames, frame):
    matches = matches.astype(jnp.int32)

    order = jnp.argsort(matches, stable=True).astype(jnp.int32)
    sorted_idx = jnp.take(matches, order)
    vals_t = jnp.take(vals.T, order, axis=1)
    vals_blk = vals_t.reshape(_D, _B // _TM, _TM).transpose(1, 0, 2)

    out_mem_t, = _scatter_mem(
        mem.T,
        vals_blk,
        sorted_idx.reshape(_B // 1024, 8, 128),
    )

    tail_lo = _NT * _TM
    m_tail = matches - tail_lo
    in_tail = m_tail >= 0
    tail_new = mem[tail_lo:].at[jnp.where(in_tail, m_tail, _TAIL)].set(
        vals, mode="drop")
    out_mem_t = lax.dynamic_update_slice(out_mem_t, tail_new.T, (0, tail_lo))

    fvals = jnp.full((_B,), frame, dtype=jnp.int32)
    out_frames, = _scatter_frames(
        frames,
        sorted_idx.reshape(_NW, _NCH, _CH),
        fvals.reshape(_NW, _NCH, _CH),
    )
    return out_mem_t.T, out_frames


# speedup vs baseline: 1.1226x; 1.1226x over previous
import jax
import jax.numpy as jnp
from jax import lax
from jax.experimental import pallas as pl
from jax.experimental.pallas import tpu as pltpu
from jax.experimental.pallas import tpu_sc as plsc
from jax._src.pallas import mpmd

_M = 1_000_000
_D = 64
_B = 16384

_NC = 2
_NS = 16
_NW = _NC * _NS

_TM = 128
_NT = _M // _TM
_TAIL = _M - _NT * _TM
_TPW = (_NT + _NW - 1) // _NW
_NBUF = 5

_OFFQ = (_NT + 2 + 1023) // 1024

_CH = 128
_NCH = _B // _NW // _CH

_mesh = plsc.VectorSubcoreMesh(
    core_axis_name="c", subcore_axis_name="s", num_cores=_NC, num_subcores=_NS)


def _splat(ref, idx_scalars):
    return plsc.load_gather(
        ref, [jnp.full((16,), i, jnp.int32) for i in idx_scalars])


def _scalar(vec):
    return jnp.squeeze(lax.slice(vec, (0,), (1,)))


def _mem_body(memT_hbm, valsT_hbm, sidx_hbm, out_memT,
              tile_v, vals_v, sidx_v, sem_in, sem_out):
    wid = lax.axis_index("s") * _NC + lax.axis_index("c")
    t0 = wid * _TPW
    nt = jnp.minimum(_TPW, _NT - t0)

    pltpu.sync_copy(sidx_hbm, sidx_v)

    def read_sidx(k):
        kc = jnp.minimum(k, _B - 1)
        return _scalar(_splat(sidx_v, (kc >> 10, (kc >> 7) & 7, kc & 127)))

    target = t0 * _TM
    lo = jnp.int32(0)
    hi = jnp.int32(_B)
    for _ in range(14):
        mid = (lo + hi) >> 1
        below = read_sidx(mid) < target
        lo = jnp.where(below, mid + 1, lo)
        hi = jnp.where(below, hi, mid)

    def col_base(t):
        return pl.multiple_of(t * _TM, _TM)

    def start_in(j, buf):
        pltpu.async_copy(memT_hbm.at[:, pl.ds(col_base(t0 + j), _TM)],
                         tile_v.at[buf], sem_in.at[buf])

    def patch(buf, t, carry):
        limit = (t + 1) * _TM
        def cond(c):
            k, _ = c
            return (k < _B) & (read_sidx(k) < limit)
        def step(c):
            k, gv = c
            g_new = k >> 7
            @pl.when(g_new != gv)
            def _():
                pltpu.sync_copy(valsT_hbm.at[g_new], vals_v)
            lv = k - g_new * _TM
            rel = read_sidx(k) - t * _TM
            for p in range(_D // 16):
                dvec = lax.iota(jnp.int32, 16) + 16 * p
                col = plsc.load_gather(
                    vals_v, [dvec, jnp.full((16,), lv, jnp.int32)])
                plsc.store_scatter(
                    tile_v.at[buf], [dvec, jnp.full((16,), rel, jnp.int32)],
                    col)
            return k + 1, g_new
        return lax.while_loop(cond, step, carry)

    def body(j, carry):
        buf = j % _NBUF
        @pl.when(j == 0)
        def _():
            start_in(0, 0)
            @pl.when(nt > 1)
            def _():
                start_in(1, 1 % _NBUF)
        nxt = (j + 2) % _NBUF
        @pl.when((j + 2 < nt) & (j >= _NBUF - 2))
        def _():
            pltpu.make_async_copy(
                tile_v.at[nxt],
                out_memT.at[:, pl.ds(col_base(t0 + j + 2 - _NBUF), _TM)],
                sem_out.at[nxt]).wait()
        @pl.when(j + 2 < nt)
        def _():
            start_in(j + 2, nxt)

        pltpu.make_async_copy(
            memT_hbm.at[:, pl.ds(col_base(t0 + j), _TM)],
            tile_v.at[buf], sem_in.at[buf]).wait()

        carry = patch(buf, t0 + j, carry)

        pltpu.async_copy(tile_v.at[buf],
                         out_memT.at[:, pl.ds(col_base(t0 + j), _TM)],
                         sem_out.at[buf])
        return carry

    carry = lax.fori_loop(0, nt, body, (lo, jnp.int32(-1)))

    for i in range(_NBUF):
        @pl.when(nt - 1 - i >= 0)
        def _():
            jj = nt - 1 - i
            pltpu.make_async_copy(
                tile_v.at[jj % _NBUF],
                out_memT.at[:, pl.ds(col_base(t0 + jj), _TM)],
                sem_out.at[jj % _NBUF]).wait()


_scatter_mem = mpmd._mpmd_map(
    [(_mesh, _mem_body)],
    [jax.ShapeDtypeStruct((_D, _M), jnp.float32)],
    scratch_types=[
        pltpu.VMEM((_NBUF, _D, _TM), jnp.float32),
        pltpu.VMEM((_D, _TM), jnp.float32),
        pltpu.VMEM((_B // 1024, 8, 128), jnp.int32),
        pltpu.SemaphoreType.DMA((_NBUF,)),
        pltpu.SemaphoreType.DMA((_NBUF,)),
    ],
    compiler_params=pltpu.CompilerParams(needs_layout_passes=False),
    name="tracker_scatter_mem",
)


def _frames_body(frames_hbm, idx_hbm, fvals_hbm, out_frames, idx_v, fv_v, sem):
    del frames_hbm
    wid = lax.axis_index("s") * _NC + lax.axis_index("c")
    pltpu.sync_copy(idx_hbm.at[wid], idx_v)
    pltpu.sync_copy(fvals_hbm.at[wid], fv_v)
    copies = []
    for j in range(_NCH):
        copies.append(
            pltpu.async_copy(fv_v.at[j], out_frames.at[idx_v.at[j]], sem))
    for cp in copies:
        cp.wait()


_scatter_frames = mpmd._mpmd_map(
    [(_mesh, _frames_body)],
    [jax.ShapeDtypeStruct((_M,), jnp.int32)],
    input_output_aliases={0: 0},
    scratch_types=[
        pltpu.VMEM((_NCH, _CH), jnp.int32),
        pltpu.VMEM((_NCH, _CH), jnp.int32),
        pltpu.SemaphoreType.DMA,
    ],
    compiler_params=pltpu.CompilerParams(use_tc_tiling_on_sc=False),
    name="tracker_scatter_frames",
)


def kernel(mem, vals, matches, frames, frame):
    matches = matches.astype(jnp.int32)

    order = jnp.argsort(matches, stable=True).astype(jnp.int32)
    sorted_idx = jnp.take(matches, order)
    vals_t = jnp.take(vals.T, order, axis=1)
    vals_blk = vals_t.reshape(_D, _B // _TM, _TM).transpose(1, 0, 2)

    out_mem_t, = _scatter_mem(
        mem.T,
        vals_blk,
        sorted_idx.reshape(_B // 1024, 8, 128),
    )

    tail_lo = _NT * _TM
    m_tail = matches - tail_lo
    in_tail = m_tail >= 0
    tail_new = mem[tail_lo:].at[jnp.where(in_tail, m_tail, _TAIL)].set(
        vals, mode="drop")
    out_mem_t = lax.dynamic_update_slice(out_mem_t, tail_new.T, (0, tail_lo))

    fvals = jnp.full((_B,), frame, dtype=jnp.int32)
    out_frames, = _scatter_frames(
        frames,
        sorted_idx.reshape(_NW, _NCH, _CH),
        fvals.reshape(_NW, _NCH, _CH),
    )
    return out_mem_t.T, out_frames


# speedup vs baseline: 1.2184x; 1.0853x over previous
import jax
import jax.numpy as jnp
from jax import lax
from jax.experimental import pallas as pl
from jax.experimental.pallas import tpu as pltpu
from jax.experimental.pallas import tpu_sc as plsc
from jax._src.pallas import mpmd

_M = 1_000_000
_D = 64
_B = 16384

_NC = 2
_NS = 16
_NW = _NC * _NS

_TM = 128
_NT = _M // _TM
_TAIL = _M - _NT * _TM
_TPW = (_NT + _NW - 1) // _NW
_NBUF = 5

_OFFQ = (_NT + 2 + 1023) // 1024

_CH = 128
_NCH = _B // _NW // _CH

_mesh = plsc.VectorSubcoreMesh(
    core_axis_name="c", subcore_axis_name="s", num_cores=_NC, num_subcores=_NS)


def _splat(ref, idx_scalars):
    return plsc.load_gather(
        ref, [jnp.full((16,), i, jnp.int32) for i in idx_scalars])


def _scalar(vec):
    return jnp.squeeze(lax.slice(vec, (0,), (1,)))


def _mem_body(memT_hbm, valsT_hbm, sidx_hbm, fvals_hbm, frames_hbm,
              out_memT, out_frames,
              tile_v, vals_v, sidx_v, fv_v, sem_in, sem_out, sem_f):
    del frames_hbm
    wid = lax.axis_index("s") * _NC + lax.axis_index("c")
    t0 = wid * _TPW
    nt = jnp.minimum(_TPW, _NT - t0)

    pltpu.sync_copy(sidx_hbm, sidx_v)
    pltpu.sync_copy(fvals_hbm, fv_v)

    frame_copies = []
    for j in range(_B // _NW // 128):
        k0 = wid * (_B // _NW) + j * 128
        frame_copies.append(
            pltpu.async_copy(fv_v.at[0],
                             out_frames.at[sidx_v.at[k0 >> 10, (k0 >> 7) & 7]],
                             sem_f))

    def read_sidx(k):
        kc = jnp.minimum(k, _B - 1)
        return _scalar(_splat(sidx_v, (kc >> 10, (kc >> 7) & 7, kc & 127)))

    target = t0 * _TM
    lo = jnp.int32(0)
    hi = jnp.int32(_B)
    for _ in range(14):
        mid = (lo + hi) >> 1
        below = read_sidx(mid) < target
        lo = jnp.where(below, mid + 1, lo)
        hi = jnp.where(below, hi, mid)

    def col_base(t):
        return pl.multiple_of(t * _TM, _TM)

    def start_in(j, buf):
        pltpu.async_copy(memT_hbm.at[:, pl.ds(col_base(t0 + j), _TM)],
                         tile_v.at[buf], sem_in.at[buf])

    def patch(buf, t, carry):
        limit = (t + 1) * _TM
        def cond(c):
            k, _ = c
            return (k < _B) & (read_sidx(k) < limit)
        def step(c):
            k, gv = c
            g_new = k >> 7
            @pl.when(g_new != gv)
            def _():
                pltpu.sync_copy(valsT_hbm.at[g_new], vals_v)
            lv = k - g_new * _TM
            rel = read_sidx(k) - t * _TM
            for p in range(_D // 16):
                dvec = lax.iota(jnp.int32, 16) + 16 * p
                col = plsc.load_gather(
                    vals_v, [dvec, jnp.full((16,), lv, jnp.int32)])
                plsc.store_scatter(
                    tile_v.at[buf], [dvec, jnp.full((16,), rel, jnp.int32)],
                    col)
            return k + 1, g_new
        return lax.while_loop(cond, step, carry)

    def body(j, carry):
        buf = j % _NBUF
        @pl.when(j == 0)
        def _():
            start_in(0, 0)
            @pl.when(nt > 1)
            def _():
                start_in(1, 1 % _NBUF)
        nxt = (j + 2) % _NBUF
        @pl.when((j + 2 < nt) & (j >= _NBUF - 2))
        def _():
            pltpu.make_async_copy(
                tile_v.at[nxt],
                out_memT.at[:, pl.ds(col_base(t0 + j + 2 - _NBUF), _TM)],
                sem_out.at[nxt]).wait()
        @pl.when(j + 2 < nt)
        def _():
            start_in(j + 2, nxt)

        pltpu.make_async_copy(
            memT_hbm.at[:, pl.ds(col_base(t0 + j), _TM)],
            tile_v.at[buf], sem_in.at[buf]).wait()

        carry = patch(buf, t0 + j, carry)

        pltpu.async_copy(tile_v.at[buf],
                         out_memT.at[:, pl.ds(col_base(t0 + j), _TM)],
                         sem_out.at[buf])
        return carry

    carry = lax.fori_loop(0, nt, body, (lo, jnp.int32(-1)))

    for i in range(_NBUF):
        @pl.when(nt - 1 - i >= 0)
        def _():
            jj = nt - 1 - i
            pltpu.make_async_copy(
                tile_v.at[jj % _NBUF],
                out_memT.at[:, pl.ds(col_base(t0 + jj), _TM)],
                sem_out.at[jj % _NBUF]).wait()

    for cp in frame_copies:
        cp.wait()


_scatter_mem = mpmd._mpmd_map(
    [(_mesh, _mem_body)],
    [jax.ShapeDtypeStruct((_D, _M), jnp.float32),
     jax.ShapeDtypeStruct((_M,), jnp.int32)],
    input_output_aliases={4: 1},
    scratch_types=[
        pltpu.VMEM((_NBUF, _D, _TM), jnp.float32),
        pltpu.VMEM((_D, _TM), jnp.float32),
        pltpu.VMEM((_B // 1024, 8, 128), jnp.int32),
        pltpu.VMEM((8, 128), jnp.int32),
        pltpu.SemaphoreType.DMA((_NBUF,)),
        pltpu.SemaphoreType.DMA((_NBUF,)),
        pltpu.SemaphoreType.DMA,
    ],
    compiler_params=pltpu.CompilerParams(needs_layout_passes=False),
    name="tracker_scatter_mem",
)


def kernel(mem, vals, matches, frames, frame):
    matches = matches.astype(jnp.int32)

    order = jnp.argsort(matches, stable=True).astype(jnp.int32)
    sorted_idx = jnp.take(matches, order)
    vals_t = jnp.take(vals.T, order, axis=1)
    vals_blk = vals_t.reshape(_D, _B // _TM, _TM).transpose(1, 0, 2)

    fvals = jnp.full((8, 128), frame, dtype=jnp.int32)
    out_mem_t, out_frames = _scatter_mem(
        mem.T,
        vals_blk,
        sorted_idx.reshape(_B // 1024, 8, 128),
        fvals,
        frames,
    )

    tail_lo = _NT * _TM
    m_tail = matches - tail_lo
    in_tail = m_tail >= 0
    tail_new = mem[tail_lo:].at[jnp.where(in_tail, m_tail, _TAIL)].set(
        vals, mode="drop")
    out_mem_t = lax.dynamic_update_slice(out_mem_t, tail_new.T, (0, tail_lo))

    return out_mem_t.T, out_frames
